# baseline (speedup 1.0000x reference)
import functools

import jax
import jax.numpy as jnp
from jax import lax
from jax.experimental import pallas as pl
from jax.experimental.pallas import tpu as pltpu
from jax.experimental.pallas import tpu_sc as plsc

_NUM_CORES = 2
_NUM_SUBCORES = 16
_LANES = 16
_NBUF = 2


def _make_sc_kernel(B, D, L, V):
    NW = _NUM_CORES * _NUM_SUBCORES
    assert B // 128 == NW
    assert L % 8 == 0 and L % _NBUF == 0
    n_groups = 128 // _LANES

    mesh = plsc.VectorSubcoreMesh(
        core_axis_name="c",
        subcore_axis_name="s",
        num_cores=_NUM_CORES,
        num_subcores=_NUM_SUBCORES,
    )

    rows_t = [pltpu.VMEM((128, D), jnp.float32) for _ in range(_NBUF)]
    bias_t = [pltpu.VMEM((128,), jnp.float32) for _ in range(_NBUF)]
    u_t = [pltpu.VMEM((D, 128), jnp.float32) for _ in range(_NBUF)]
    out_t = [pltpu.VMEM((128,), jnp.float32) for _ in range(_NBUF)]
    semi_t = [pltpu.SemaphoreType.DMA for _ in range(_NBUF)]
    semo_t = [pltpu.SemaphoreType.DMA for _ in range(_NBUF)]

    @functools.partial(
        pl.kernel,
        out_type=jax.ShapeDtypeStruct((B * L,), jnp.float32),
        mesh=mesh,
        compiler_params=pltpu.CompilerParams(
            needs_layout_passes=False, use_tc_tiling_on_sc=False),
        scratch_types=[pltpu.VMEM((L, 128), jnp.int32)]
        + rows_t + bias_t + u_t + out_t + semi_t + semo_t,
    )
    def sc_kernel(u5_hbm, tgt_hbm, emb_hbm, bias_hbm, out_hbm,
                  idx_v, *scr):
        rows = scr[0:_NBUF]
        biasv = scr[_NBUF:2 * _NBUF]
        uv = scr[2 * _NBUF:3 * _NBUF]
        outv = scr[3 * _NBUF:4 * _NBUF]
        semi = scr[4 * _NBUF:5 * _NBUF]
        semo = scr[5 * _NBUF:6 * _NBUF]

        wid = lax.axis_index("s") * _NUM_CORES + lax.axis_index("c")
        pltpu.sync_copy(tgt_hbm.at[:, pl.ds(wid * 128, 128)], idx_v)

        lane = lax.iota(jnp.int32, 16)

        def issue(l, k):
            pltpu.async_copy(emb_hbm.at[idx_v.at[l]], rows[k], semi[k])
            pltpu.async_copy(bias_hbm.at[idx_v.at[l]], biasv[k], semi[k])
            pltpu.async_copy(u5_hbm.at[:, l // 8, wid, l % 8, :],
                             uv[k], semi[k])

        def drain_in(k):
            pltpu.make_async_copy(emb_hbm.at[pl.ds(0, 128)],
                                  rows[k], semi[k]).wait()
            pltpu.make_async_copy(bias_hbm.at[pl.ds(0, 128)],
                                  biasv[k], semi[k]).wait()
            pltpu.make_async_copy(u5_hbm.at[:, 0, 0, 0, :],
                                  uv[k], semi[k]).wait()

        def drain_out(k):
            pltpu.make_async_copy(out_hbm.at[pl.ds(0, 128)],
                                  outv[k], semo[k]).wait()

        def body(i, carry):
            for k in range(_NBUF):
                l = _NBUF * i + k
                rows_v, bias_v, u_v, out_v = rows[k], biasv[k], uv[k], outv[k]

                @pl.when(l + (_NBUF - 1) < L)
                def _():
                    issue(l + (_NBUF - 1), (k + _NBUF - 1) % _NBUF)

                drain_in(k)

                @pl.when(l >= _NBUF)
                def _():
                    drain_out(k)

                for grp in range(n_groups):
                    s = grp * _LANES
                    pos = s + lane
                    acc = bias_v[pl.ds(s, _LANES)]
                    for d in range(D):
                        uvec = u_v[d, pl.ds(s, _LANES)]
                        dvec = jnp.full((16,), d, dtype=jnp.int32)
                        evec = plsc.load_gather(rows_v, [pos, dvec])
                        acc = acc + uvec * evec
                    out_v[pl.ds(s, _LANES)] = acc
                pltpu.async_copy(out_v,
                                 out_hbm.at[pl.ds(l * B + wid * 128, 128)],
                                 semo[k])
            return carry

        for l0 in range(_NBUF - 1):
            issue(l0, l0)
        lax.fori_loop(0, L // _NBUF, body, 0, unroll=False)
        for k in range(_NBUF):
            drain_out(k)

    return sc_kernel


def kernel(user_representations, targets, item_emb, item_bias):
    B, D, L = user_representations.shape
    V = item_emb.shape[0]
    u5 = jnp.transpose(user_representations, (1, 2, 0))
    u5 = u5.reshape(D, L // 8, 8, B // 128, 128)
    u5 = jnp.transpose(u5, (0, 1, 3, 2, 4))
    tgt_t = jnp.transpose(targets, (1, 0))
    bias1d = item_bias.reshape((V,))
    sc = _make_sc_kernel(B, D, L, V)
    flat = sc(u5, tgt_t, item_emb, bias1d)
    return jnp.transpose(flat.reshape(L, B), (1, 0))
